# baseline (speedup 1.0000x reference)
import jax
import jax.numpy as jnp
from jax import lax
from jax.experimental import pallas as pl
from jax.experimental.pallas import tpu as pltpu
from jax.experimental.pallas import tpu_sc as plsc

N_POINTS = 262144
IN_DIM = 3
N_LEVELS = 16
F_PER_LEVEL = 2
LOG2_T = 19
T = 1 << LOG2_T
BASE_RES = 16
PER_LEVEL_SCALE = 1.3819128799
P1 = 2654435761
P2 = 805459861
OUT_DIM = IN_DIM + N_LEVELS * F_PER_LEVEL


def _res_floor(l):
    import numpy as np
    return int(np.floor(BASE_RES * (PER_LEVEL_SCALE ** l)))


RES = [_res_floor(l) for l in range(N_LEVELS)]
DENSE = [(r + 1) ** 3 <= T for r in RES]

NW = 32
PTS_PER_W = N_POINTS // NW
C = 128
NCHUNK = PTS_PER_W // C
NV = C // 16


def _body(x0, x1, x2, table, out, xbuf, fracbuf, idxbuf, featbuf, outbuf, sem0, sem1):
    xs = (x0, x1, x2)
    cid = lax.axis_index("c")
    sid = lax.axis_index("s")
    wid = sid * 2 + cid
    iota = lax.iota(jnp.int32, 16)
    iota35 = iota * OUT_DIM

    def chunk_body(ch, carry):
        base = wid * PTS_PER_W + ch * C
        for d in range(IN_DIM):
            pltpu.sync_copy(xs[d].at[pl.ds(base, C)], xbuf.at[pl.ds(d * C, C)])

        def xcol_body(i, c2):
            fo = i * (16 * OUT_DIM)
            for d in range(IN_DIM):
                v = xbuf[pl.ds(d * C + i * 16, 16)]
                plsc.store_scatter(outbuf, [iota35 + (fo + d)], v)
            return c2
        lax.fori_loop(0, NV, xcol_body, 0)

        def passA(l):
            p = l & 1
            resf = jnp.float32(RES[l])

            def body_i(i, c2):
                o = i * 16
                ip = []
                for d in range(IN_DIM):
                    pos = xbuf[pl.ds(d * C + o, 16)] * resf
                    ipd = pos.astype(jnp.int32)
                    fracbuf[pl.ds((p * IN_DIM + d) * C + o, 16)] = (
                        pos - ipd.astype(jnp.float32))
                    ip.append(ipd)
                if DENSE[l]:
                    s = RES[l] + 1
                    h0 = (ip[0], ip[0] + 1)
                    h1 = (ip[1] * s, ip[1] * s + s)
                    b2 = ip[2] * (s * s) + (l * T)
                    h2 = (b2, b2 + s * s)
                    t01 = [h0[0] + h1[0], h0[1] + h1[0], h0[0] + h1[1], h0[1] + h1[1]]
                    for c in range(8):
                        idx = t01[c & 3] + h2[(c >> 2) & 1]
                        e = idx + idx
                        idxbuf[pl.ds((p * 16 + 2 * c) * C + o, 16)] = e
                        idxbuf[pl.ds((p * 16 + 2 * c + 1) * C + o, 16)] = e + 1
                else:
                    u = [plsc.bitcast(v, jnp.uint32) for v in ip]
                    h0 = (u[0], u[0] + jnp.uint32(1))
                    m1 = u[1] * jnp.uint32(P1)
                    h1 = (m1, m1 + jnp.uint32(P1))
                    m2 = u[2] * jnp.uint32(P2)
                    h2 = (m2, m2 + jnp.uint32(P2))
                    t01 = [h0[0] ^ h1[0], h0[1] ^ h1[0], h0[0] ^ h1[1], h0[1] ^ h1[1]]
                    mask = jnp.uint32(T - 1)
                    for c in range(8):
                        hv = t01[c & 3] ^ h2[(c >> 2) & 1]
                        idx = plsc.bitcast(hv & mask, jnp.int32) + (l * T)
                        e = idx + idx
                        idxbuf[pl.ds((p * 16 + 2 * c) * C + o, 16)] = e
                        idxbuf[pl.ds((p * 16 + 2 * c + 1) * C + o, 16)] = e + 1
                return c2
            lax.fori_loop(0, NV, body_i, 0)

        def fire(l):
            p = l & 1
            sem = sem0 if p == 0 else sem1
            handles = []
            for k in range(2 * 8):
                q = (p * 16 + k) * C
                handles.append(
                    pltpu.async_copy(table.at[idxbuf.at[pl.ds(q, C)]],
                                     featbuf.at[pl.ds(q, C)], sem))
            return handles

        def passB(l):
            p = l & 1
            col = IN_DIM + 2 * l

            def body_i(i, c2):
                o = i * 16
                fo = i * (16 * OUT_DIM)
                f0 = fracbuf[pl.ds((p * IN_DIM + 0) * C + o, 16)]
                f1 = fracbuf[pl.ds((p * IN_DIM + 1) * C + o, 16)]
                f2 = fracbuf[pl.ds((p * IN_DIM + 2) * C + o, 16)]
                g0 = (1.0 - f0, f0)
                g1 = (1.0 - f1, f1)
                g2 = (1.0 - f2, f2)
                w01 = [g0[0] * g1[0], g0[1] * g1[0], g0[0] * g1[1], g0[1] * g1[1]]
                acc0 = jnp.zeros((16,), jnp.float32)
                acc1 = jnp.zeros((16,), jnp.float32)
                for c in range(8):
                    w = w01[c & 3] * g2[(c >> 2) & 1]
                    acc0 = acc0 + w * featbuf[pl.ds((p * 16 + 2 * c) * C + o, 16)]
                    acc1 = acc1 + w * featbuf[pl.ds((p * 16 + 2 * c + 1) * C + o, 16)]
                plsc.store_scatter(outbuf, [iota35 + (fo + col)], acc0)
                plsc.store_scatter(outbuf, [iota35 + (fo + col + 1)], acc1)
                return c2
            lax.fori_loop(0, NV, body_i, 0)

        passA(0)
        pending = fire(0)
        for l in range(1, N_LEVELS + 1):
            if l < N_LEVELS:
                passA(l)
                nxt = fire(l)
            for h in pending:
                h.wait()
            passB(l - 1)
            if l < N_LEVELS:
                pending = nxt

        pltpu.sync_copy(outbuf, out.at[pl.ds(base * OUT_DIM, C * OUT_DIM)])
        return carry

    lax.fori_loop(0, NCHUNK, chunk_body, 0)


@jax.jit
def kernel(x, grid):
    x0, x1, x2 = x[:, 0], x[:, 1], x[:, 2]
    table = grid.reshape(N_LEVELS * T * F_PER_LEVEL)
    mesh = plsc.VectorSubcoreMesh(core_axis_name="c", subcore_axis_name="s")
    f = pl.kernel(
        _body,
        out_type=jax.ShapeDtypeStruct((N_POINTS * OUT_DIM,), jnp.float32),
        mesh=mesh,
        compiler_params=pltpu.CompilerParams(needs_layout_passes=False),
        scratch_types=[
            pltpu.VMEM((IN_DIM * C,), jnp.float32),
            pltpu.VMEM((2 * IN_DIM * C,), jnp.float32),
            pltpu.VMEM((2 * 2 * 8 * C,), jnp.int32),
            pltpu.VMEM((2 * 2 * 8 * C,), jnp.float32),
            pltpu.VMEM((C * OUT_DIM,), jnp.float32),
            pltpu.SemaphoreType.DMA,
            pltpu.SemaphoreType.DMA,
        ],
    )
    return f(x0, x1, x2, table).reshape(N_POINTS, OUT_DIM)
